# baseline (speedup 1.0000x reference)
import jax
import jax.numpy as jnp
from jax import lax
from jax.experimental import pallas as pl
from jax.experimental.pallas import tpu as pltpu
from jax.experimental.pallas import tpu_sc as plsc

N = 10000
E = 320000
D = 128
ACT = 16
G = N // ACT

NC = 2
NS = 16
NT = NC * NS
CHUNK = 128
CPT = 80
CH = 64
CPA = 160
EP = NT * CPA * CH
PAD_ROWS = 240
NP = N + PAD_ROWS
RPT = NP // NS
SUPERS = CPA // 2

_mesh = plsc.VectorSubcoreMesh(core_axis_name="c", subcore_axis_name="s")


def _deg_body(dst3, out, acc, dstb, ones_v, zbuf, dsem):
    c = lax.axis_index("c")
    s = lax.axis_index("s")
    w = s * NC + c
    o16 = jnp.ones((16,), jnp.float32)
    z16 = jnp.zeros((16,), jnp.float32)
    for jj in range(CHUNK // 16):
        ones_v[pl.ds(jj * 16, 16)] = o16
    for jj in range(RPT // 16):
        zbuf[pl.ds(jj * 16, 16)] = z16
    pltpu.sync_copy(zbuf, acc.at[pl.ds(s * RPT, RPT)])
    plsc.subcore_barrier()
    pltpu.sync_copy(dst3.at[w], dstb)

    def body(t, carry):
        for k in range(8):
            pltpu.async_copy(ones_v, acc.at[dstb.at[t * 8 + k]], dsem, add=True)
        for k in range(8):
            pltpu.make_async_copy(ones_v, acc.at[dstb.at[t * 8 + k]], dsem).wait()
        return carry

    lax.fori_loop(0, CPT // 8, body, 0)
    plsc.subcore_barrier()
    pltpu.sync_copy(acc.at[pl.ds(s * RPT, RPT)], out.at[c, pl.ds(s * RPT, RPT)])


_deg_kernel = pl.kernel(
    _deg_body,
    out_type=jax.ShapeDtypeStruct((NC, NP), jnp.float32),
    mesh=_mesh,
    scratch_types=[
        pltpu.VMEM_SHARED((NP,), jnp.float32),
        pltpu.VMEM((CPT, CHUNK), jnp.int32),
        pltpu.VMEM((CHUNK,), jnp.float32),
        pltpu.VMEM((RPT,), jnp.float32),
        pltpu.SemaphoreType.DMA,
    ],
)


def _agg_body(y_h, src2, dst3, out, acc, srcb, dstb, rows0, rows1,
              gsem0, gsem1, ssem0, ssem1):
    c = lax.axis_index("c")
    s = lax.axis_index("s")
    w = s * NC + c
    z16 = jnp.zeros((16,), jnp.float32)

    def fire_g(it, buf, sem):
        pltpu.async_copy(y_h.at[srcb.at[pl.ds(it * CH, CH)]], buf, sem)

    def wait_g(it, buf, sem):
        pltpu.make_async_copy(y_h.at[srcb.at[pl.ds(it * CH, CH)]], buf,
                              sem).wait()

    def fire_s(it, buf, sem):
        pltpu.async_copy(buf, acc.at[dstb.at[it]], sem, add=True)

    def wait_s(it, buf, sem):
        pltpu.make_async_copy(buf, acc.at[dstb.at[it]], sem).wait()

    def zrow(i, carry):
        for jj in range(D // 16):
            rows0[i, pl.ds(jj * 16, 16)] = z16
        return carry

    lax.fori_loop(0, CH, zrow, 0)
    for k in range(RPT // CH):
        pltpu.sync_copy(rows0, acc.at[pl.ds(s * RPT + k * CH, CH)])
    plsc.subcore_barrier()

    pltpu.sync_copy(src2.at[w], srcb)
    pltpu.sync_copy(dst3.at[w], dstb)

    fire_g(0, rows0, gsem0)

    def body(t, carry):
        it0 = 2 * t
        it1 = it0 + 1

        @pl.when(t > 0)
        def _():
            wait_s(it1 - 2, rows1, ssem1)

        fire_g(it1, rows1, gsem1)
        wait_g(it0, rows0, gsem0)
        fire_s(it0, rows0, ssem0)
        wait_g(it1, rows1, gsem1)
        fire_s(it1, rows1, ssem1)
        wait_s(it0, rows0, ssem0)

        @pl.when(t < SUPERS - 1)
        def _():
            fire_g(it0 + 2, rows0, gsem0)

        return carry

    lax.fori_loop(0, SUPERS, body, 0)
    wait_s(CPA - 1, rows1, ssem1)
    plsc.subcore_barrier()
    pltpu.sync_copy(acc.at[pl.ds(s * RPT, RPT)], out.at[c, pl.ds(s * RPT, RPT)])


_agg_kernel = pl.kernel(
    _agg_body,
    out_type=jax.ShapeDtypeStruct((NC, NP, D), jnp.float32),
    mesh=_mesh,
    scratch_types=[
        pltpu.VMEM_SHARED((NP, D), jnp.float32),
        pltpu.VMEM((CPA * CH,), jnp.int32),
        pltpu.VMEM((CPA, CH), jnp.int32),
        pltpu.VMEM((CH, D), jnp.float32),
        pltpu.VMEM((CH, D), jnp.float32),
        pltpu.SemaphoreType.DMA,
        pltpu.SemaphoreType.DMA,
        pltpu.SemaphoreType.DMA,
        pltpu.SemaphoreType.DMA,
    ],
)


def _mm_body(state_ref, w1_ref, xw_ref):
    xw_ref[...] = jnp.dot(state_ref[...], w1_ref[...],
                          preferred_element_type=jnp.float32)


def _scale_body(xw_ref, degr_ref, y_ref, dinv_ref):
    deg = degr_ref[0:N] + degr_ref[NP:NP + N] + 1.0
    dinv = lax.rsqrt(deg)
    y_ref[...] = xw_ref[...] * dinv
    dinv_ref[...] = dinv


def _head_body(aggr_ref, y_ref, dinv_ref, state_ref, b1_ref,
               wl1_ref, bl1_ref, wl2_ref, bl2_ref, wl3_ref, bl3_ref, out_ref):
    agg = aggr_ref[0:N] + aggr_ref[NP:NP + N] + y_ref[...]
    t = jnp.maximum(dinv_ref[...] * agg + b1_ref[...], 0.0) + state_ref[...]
    h = jnp.sum(t.reshape(G, ACT, D), axis=1)
    h = jnp.maximum(jnp.dot(h, wl1_ref[...], preferred_element_type=jnp.float32)
                    + bl1_ref[...], 0.0)
    h = jnp.maximum(jnp.dot(h, wl2_ref[...], preferred_element_type=jnp.float32)
                    + bl2_ref[...], 0.0)
    out_ref[...] = jnp.dot(h, wl3_ref[...], preferred_element_type=jnp.float32) \
        + bl3_ref[...]


_tc_params = pltpu.CompilerParams(vmem_limit_bytes=100 * 1024 * 1024)


@jax.jit
def kernel(state, edge_index, W1, b1, Wl1, bl1, Wl2, bl2, Wl3, bl3):
    pad = EP - E
    ar = jnp.arange(pad, dtype=jnp.int32)
    src_p = jnp.concatenate([edge_index[0], ar % 2048])
    dst_p = jnp.concatenate([edge_index[1], N + (ar % PAD_ROWS)])
    src2 = src_p.reshape(NT, CPA * CH)
    dst3 = dst_p.reshape(NT, CPA, CH)

    degp = _deg_kernel(dst_p.reshape(NT, CPT, CHUNK))
    degr = degp.reshape(NC * NP, 1)

    xw = pl.pallas_call(
        _mm_body,
        out_shape=jax.ShapeDtypeStruct((N, D), jnp.float32),
        compiler_params=_tc_params,
    )(state, W1)

    y, dinv = pl.pallas_call(
        _scale_body,
        out_shape=[jax.ShapeDtypeStruct((N, D), jnp.float32),
                   jax.ShapeDtypeStruct((N, 1), jnp.float32)],
        compiler_params=_tc_params,
    )(xw, degr)

    aggp = _agg_kernel(y, src2, dst3)
    aggr = aggp.reshape(NC * NP, D)

    out = pl.pallas_call(
        _head_body,
        out_shape=jax.ShapeDtypeStruct((G, 1), jnp.float32),
        compiler_params=_tc_params,
    )(aggr, y, dinv, state, b1.reshape(1, D),
      Wl1, bl1.reshape(1, 64), Wl2, bl2.reshape(1, 64), Wl3, bl3.reshape(1, 1))
    return out[:, 0]


# speedup vs baseline: 1.1097x; 1.1097x over previous
import jax
import jax.numpy as jnp
from jax import lax
from jax.experimental import pallas as pl
from jax.experimental.pallas import tpu as pltpu
from jax.experimental.pallas import tpu_sc as plsc

N = 10000
E = 320000
D = 128
ACT = 16
G = N // ACT

NC = 2
NS = 16
NT = NC * NS
CHUNK = 128
CPT = 80
EPT = CPT * CHUNK
EP = NT * EPT
PAD_ROWS = 240
NP = N + PAD_ROWS
RPT = NP // NS
SUPERS = CPT // 2
WSUP = 4

_mesh = plsc.VectorSubcoreMesh(core_axis_name="c", subcore_axis_name="s")


def _deg_body(dst3, out, acc, dstb, ones_v, zbuf, dsem):
    c = lax.axis_index("c")
    s = lax.axis_index("s")
    w = s * NC + c
    o16 = jnp.ones((16,), jnp.float32)
    z16 = jnp.zeros((16,), jnp.float32)
    for jj in range(CHUNK // 16):
        ones_v[pl.ds(jj * 16, 16)] = o16
    for jj in range(RPT // 16):
        zbuf[pl.ds(jj * 16, 16)] = z16
    pltpu.sync_copy(zbuf, acc.at[pl.ds(s * RPT, RPT)])
    plsc.subcore_barrier()
    pltpu.sync_copy(dst3.at[w], dstb)

    def body(t, carry):
        for k in range(8):
            pltpu.async_copy(ones_v, acc.at[dstb.at[t * 8 + k]], dsem, add=True)
        for k in range(8):
            pltpu.make_async_copy(ones_v, acc.at[dstb.at[t * 8 + k]], dsem).wait()
        return carry

    lax.fori_loop(0, CPT // 8, body, 0)
    plsc.subcore_barrier()
    pltpu.sync_copy(acc.at[pl.ds(s * RPT, RPT)], out.at[c, pl.ds(s * RPT, RPT)])


_deg_kernel = pl.kernel(
    _deg_body,
    out_type=jax.ShapeDtypeStruct((NC, NP), jnp.float32),
    mesh=_mesh,
    scratch_types=[
        pltpu.VMEM_SHARED((NP,), jnp.float32),
        pltpu.VMEM((CPT, CHUNK), jnp.int32),
        pltpu.VMEM((CHUNK,), jnp.float32),
        pltpu.VMEM((RPT,), jnp.float32),
        pltpu.SemaphoreType.DMA,
    ],
)


def _agg_body(y_h, src2, dst3, out, acc, srcb, dstw, rows0, rows1,
              gsem0, gsem1, ssem0, ssem1, isem):
    c = lax.axis_index("c")
    s = lax.axis_index("s")
    w = s * NC + c
    z16 = jnp.zeros((16,), jnp.float32)

    def fire_g(it, buf, sem):
        pltpu.async_copy(y_h.at[srcb.at[pl.ds(it * CHUNK, CHUNK)]], buf, sem)

    def wait_g(it, buf, sem):
        pltpu.make_async_copy(y_h.at[srcb.at[pl.ds(it * CHUNK, CHUNK)]], buf,
                              sem).wait()

    def fire_s(it, buf, sem):
        pltpu.async_copy(buf, acc.at[dstw.at[it % 16]], sem, add=True)

    def wait_s(it, buf, sem):
        pltpu.make_async_copy(buf, acc.at[dstw.at[it % 16]], sem).wait()

    def zrow(i, carry):
        for jj in range(D // 16):
            rows0[i, pl.ds(jj * 16, 16)] = z16
        return carry

    lax.fori_loop(0, CHUNK, zrow, 0)
    for k in range(RPT // CHUNK):
        pltpu.sync_copy(rows0, acc.at[pl.ds(s * RPT + k * CHUNK, CHUNK)])
    plsc.subcore_barrier()

    pltpu.sync_copy(src2.at[w], srcb)
    pltpu.sync_copy(dst3.at[w, pl.ds(0, 8)], dstw.at[pl.ds(0, 8)])
    pltpu.async_copy(dst3.at[w, pl.ds(8, 8)], dstw.at[pl.ds(8, 8)], isem)

    fire_g(0, rows0, gsem0)

    def body(t, carry):
        it0 = 2 * t
        it1 = it0 + 1

        @pl.when(t > 0)
        def _():
            wait_s(it1 - 2, rows1, ssem1)

        @pl.when(jnp.logical_and(t % WSUP == 0, t > 0))
        def _():
            pltpu.make_async_copy(dst3.at[w, pl.ds(0, 8)],
                                  dstw.at[pl.ds(0, 8)], isem).wait()

        @pl.when(jnp.logical_and(t % WSUP == 0, t < SUPERS - WSUP))
        def _():
            nw = t // WSUP + 1
            half = (nw % 2) * 8
            pltpu.async_copy(dst3.at[w, pl.ds(nw * 8, 8)],
                             dstw.at[pl.ds(half, 8)], isem)

        fire_g(it1, rows1, gsem1)
        wait_g(it0, rows0, gsem0)
        fire_s(it0, rows0, ssem0)
        wait_g(it1, rows1, gsem1)
        fire_s(it1, rows1, ssem1)
        wait_s(it0, rows0, ssem0)

        @pl.when(t < SUPERS - 1)
        def _():
            fire_g(it0 + 2, rows0, gsem0)

        return carry

    lax.fori_loop(0, SUPERS, body, 0)
    wait_s(CPT - 1, rows1, ssem1)
    plsc.subcore_barrier()
    pltpu.sync_copy(acc.at[pl.ds(s * RPT, RPT)], out.at[c, pl.ds(s * RPT, RPT)])


_agg_kernel = pl.kernel(
    _agg_body,
    out_type=jax.ShapeDtypeStruct((NC, NP, D), jnp.float32),
    mesh=_mesh,
    scratch_types=[
        pltpu.VMEM_SHARED((NP, D), jnp.float32),
        pltpu.VMEM((EPT,), jnp.int32),
        pltpu.VMEM((16, CHUNK), jnp.int32),
        pltpu.VMEM((CHUNK, D), jnp.float32),
        pltpu.VMEM((CHUNK, D), jnp.float32),
        pltpu.SemaphoreType.DMA,
        pltpu.SemaphoreType.DMA,
        pltpu.SemaphoreType.DMA,
        pltpu.SemaphoreType.DMA,
        pltpu.SemaphoreType.DMA,
    ],
)


def _pre_body(state_ref, w1_ref, degr_ref, y_ref, dinv_ref):
    deg = degr_ref[0:N] + degr_ref[NP:NP + N] + 1.0
    dinv = lax.rsqrt(deg)
    xw = jnp.dot(state_ref[...], w1_ref[...], preferred_element_type=jnp.float32)
    y_ref[...] = xw * dinv
    dinv_ref[...] = dinv


def _head_body(aggr_ref, y_ref, dinv_ref, state_ref, b1_ref,
               wl1_ref, bl1_ref, wl2_ref, bl2_ref, wl3_ref, bl3_ref, out_ref):
    agg = aggr_ref[0:N] + aggr_ref[NP:NP + N] + y_ref[...]
    t = jnp.maximum(dinv_ref[...] * agg + b1_ref[...], 0.0) + state_ref[...]
    h = jnp.sum(t.reshape(G, ACT, D), axis=1)
    h = jnp.maximum(jnp.dot(h, wl1_ref[...], preferred_element_type=jnp.float32)
                    + bl1_ref[...], 0.0)
    h = jnp.maximum(jnp.dot(h, wl2_ref[...], preferred_element_type=jnp.float32)
                    + bl2_ref[...], 0.0)
    out_ref[...] = jnp.dot(h, wl3_ref[...], preferred_element_type=jnp.float32) \
        + bl3_ref[...]


_tc_params = pltpu.CompilerParams(vmem_limit_bytes=100 * 1024 * 1024)


@jax.jit
def kernel(state, edge_index, W1, b1, Wl1, bl1, Wl2, bl2, Wl3, bl3):
    pad = EP - E
    ar = jnp.arange(pad, dtype=jnp.int32)
    src_p = jnp.concatenate([edge_index[0], ar % 2048])
    dst_p = jnp.concatenate([edge_index[1], N + (ar % PAD_ROWS)])
    src2 = src_p.reshape(NT, EPT)
    dst3 = dst_p.reshape(NT, CPT, CHUNK)

    degp = _deg_kernel(dst3)
    degr = degp.reshape(NC * NP, 1)

    y, dinv = pl.pallas_call(
        _pre_body,
        out_shape=[jax.ShapeDtypeStruct((N, D), jnp.float32),
                   jax.ShapeDtypeStruct((N, 1), jnp.float32)],
        compiler_params=_tc_params,
    )(state, W1, degr)

    aggp = _agg_kernel(y, src2, dst3)
    aggr = aggp.reshape(NC * NP, D)

    out = pl.pallas_call(
        _head_body,
        out_shape=jax.ShapeDtypeStruct((G, 1), jnp.float32),
        compiler_params=_tc_params,
    )(aggr, y, dinv, state, b1.reshape(1, D),
      Wl1, bl1.reshape(1, 64), Wl2, bl2.reshape(1, 64), Wl3, bl3.reshape(1, 1))
    return out[:, 0]


# speedup vs baseline: 1.2263x; 1.1050x over previous
import jax
import jax.numpy as jnp
from jax import lax
from jax.experimental import pallas as pl
from jax.experimental.pallas import tpu as pltpu
from jax.experimental.pallas import tpu_sc as plsc

N = 10000
E = 320000
D = 128
ACT = 16
G = N // ACT

NC = 2
NS = 16
NT = NC * NS
CHUNK = 128
CPT = 80
EPT = CPT * CHUNK
EP = NT * EPT
PAD_ROWS = 240
NP = N + PAD_ROWS
RPT = NP // NS
SUPERS = CPT // 2
WSUP = 4

_mesh = plsc.VectorSubcoreMesh(core_axis_name="c", subcore_axis_name="s")


def _deg_body(dst3, out, acc, dstb, ones_v, zbuf, dsem):
    c = lax.axis_index("c")
    s = lax.axis_index("s")
    w = s * NC + c
    o16 = jnp.ones((16,), jnp.float32)
    z16 = jnp.zeros((16,), jnp.float32)
    for jj in range(CHUNK // 16):
        ones_v[pl.ds(jj * 16, 16)] = o16
    for jj in range(RPT // 16):
        zbuf[pl.ds(jj * 16, 16)] = z16
    pltpu.sync_copy(zbuf, acc.at[pl.ds(s * RPT, RPT)])
    plsc.subcore_barrier()
    pltpu.sync_copy(dst3.at[w], dstb)

    def body(t, carry):
        for k in range(8):
            pltpu.async_copy(ones_v, acc.at[dstb.at[t * 8 + k]], dsem, add=True)
        for k in range(8):
            pltpu.make_async_copy(ones_v, acc.at[dstb.at[t * 8 + k]], dsem).wait()
        return carry

    lax.fori_loop(0, CPT // 8, body, 0)
    plsc.subcore_barrier()
    pltpu.sync_copy(acc.at[pl.ds(s * RPT, RPT)], out.at[c, pl.ds(s * RPT, RPT)])


_deg_kernel = pl.kernel(
    _deg_body,
    out_type=jax.ShapeDtypeStruct((NC, NP), jnp.float32),
    mesh=_mesh,
    scratch_types=[
        pltpu.VMEM_SHARED((NP,), jnp.float32),
        pltpu.VMEM((CPT, CHUNK), jnp.int32),
        pltpu.VMEM((CHUNK,), jnp.float32),
        pltpu.VMEM((RPT,), jnp.float32),
        pltpu.SemaphoreType.DMA,
    ],
)


def _agg_body(y_h, src2, dst3, out, acc, srcb, dstw, rows0, rows1,
              gsem0, gsem1, ssem0, ssem1, isem):
    c = lax.axis_index("c")
    s = lax.axis_index("s")
    w = s * NC + c
    z16 = jnp.zeros((16,), jnp.float32)

    def fire_g(it, buf, sem):
        pltpu.async_copy(y_h.at[srcb.at[pl.ds(it * CHUNK, CHUNK)]], buf, sem)

    def wait_g(it, buf, sem):
        pltpu.make_async_copy(y_h.at[srcb.at[pl.ds(it * CHUNK, CHUNK)]], buf,
                              sem).wait()

    def fire_s(it, buf, sem):
        pltpu.async_copy(buf, acc.at[dstw.at[it % 16]], sem, add=True)

    def wait_s(it, buf, sem):
        pltpu.make_async_copy(buf, acc.at[dstw.at[it % 16]], sem).wait()

    def zrow(i, carry):
        for jj in range(D // 16):
            rows0[i, pl.ds(jj * 16, 16)] = z16
        return carry

    lax.fori_loop(0, CHUNK, zrow, 0)
    for k in range(RPT // CHUNK):
        pltpu.sync_copy(rows0, acc.at[pl.ds(s * RPT + k * CHUNK, CHUNK)])
    plsc.subcore_barrier()

    pltpu.sync_copy(src2.at[w], srcb)
    pltpu.sync_copy(dst3.at[w, pl.ds(0, 8)], dstw.at[pl.ds(0, 8)])

    fire_g(0, rows0, gsem0)

    def body(t, carry):
        it0 = 2 * t
        it1 = it0 + 1

        @pl.when(t > 0)
        def _():
            wait_s(it1 - 2, rows1, ssem1)

        @pl.when(jnp.logical_and(t % WSUP == 0, t > 0))
        def _():
            cw = t // WSUP
            half = (cw % 2) * 8
            pltpu.sync_copy(dst3.at[w, pl.ds(cw * 8, 8)],
                            dstw.at[pl.ds(half, 8)])

        fire_g(it1, rows1, gsem1)
        wait_g(it0, rows0, gsem0)
        fire_s(it0, rows0, ssem0)
        wait_g(it1, rows1, gsem1)
        fire_s(it1, rows1, ssem1)
        wait_s(it0, rows0, ssem0)

        @pl.when(t < SUPERS - 1)
        def _():
            fire_g(it0 + 2, rows0, gsem0)

        return carry

    lax.fori_loop(0, SUPERS, body, 0)
    wait_s(CPT - 1, rows1, ssem1)
    plsc.subcore_barrier()
    pltpu.sync_copy(acc.at[pl.ds(s * RPT, RPT)], out.at[c, pl.ds(s * RPT, RPT)])


_agg_kernel = pl.kernel(
    _agg_body,
    out_type=jax.ShapeDtypeStruct((NC, NP, D), jnp.float32),
    mesh=_mesh,
    scratch_types=[
        pltpu.VMEM_SHARED((NP, D), jnp.float32),
        pltpu.VMEM((EPT,), jnp.int32),
        pltpu.VMEM((16, CHUNK), jnp.int32),
        pltpu.VMEM((CHUNK, D), jnp.float32),
        pltpu.VMEM((CHUNK, D), jnp.float32),
        pltpu.SemaphoreType.DMA,
        pltpu.SemaphoreType.DMA,
        pltpu.SemaphoreType.DMA,
        pltpu.SemaphoreType.DMA,
        pltpu.SemaphoreType.DMA,
    ],
)


def _pre_body(state_ref, w1_ref, degr_ref, y_ref, dinv_ref):
    deg = degr_ref[0:N] + degr_ref[NP:NP + N] + 1.0
    dinv = lax.rsqrt(deg)
    xw = jnp.dot(state_ref[...], w1_ref[...], preferred_element_type=jnp.float32)
    y_ref[...] = xw * dinv
    dinv_ref[...] = dinv


def _head_body(aggr_ref, y_ref, dinv_ref, state_ref, b1_ref,
               wl1_ref, bl1_ref, wl2_ref, bl2_ref, wl3_ref, bl3_ref, out_ref):
    agg = aggr_ref[0:N] + aggr_ref[NP:NP + N] + y_ref[...]
    t = jnp.maximum(dinv_ref[...] * agg + b1_ref[...], 0.0) + state_ref[...]
    h = jnp.sum(t.reshape(G, ACT, D), axis=1)
    h = jnp.maximum(jnp.dot(h, wl1_ref[...], preferred_element_type=jnp.float32)
                    + bl1_ref[...], 0.0)
    h = jnp.maximum(jnp.dot(h, wl2_ref[...], preferred_element_type=jnp.float32)
                    + bl2_ref[...], 0.0)
    out_ref[...] = jnp.dot(h, wl3_ref[...], preferred_element_type=jnp.float32) \
        + bl3_ref[...]


_tc_params = pltpu.CompilerParams(vmem_limit_bytes=100 * 1024 * 1024)


@jax.jit
def kernel(state, edge_index, W1, b1, Wl1, bl1, Wl2, bl2, Wl3, bl3):
    pad = EP - E
    ar = jnp.arange(pad, dtype=jnp.int32)
    src_p = jnp.concatenate([edge_index[0], ar % 2048])
    dst_p = jnp.concatenate([edge_index[1], N + (ar % PAD_ROWS)])
    src2 = src_p.reshape(NT, EPT)
    dst3 = dst_p.reshape(NT, CPT, CHUNK)

    degp = _deg_kernel(dst3)
    degr = degp.reshape(NC * NP, 1)

    y, dinv = pl.pallas_call(
        _pre_body,
        out_shape=[jax.ShapeDtypeStruct((N, D), jnp.float32),
                   jax.ShapeDtypeStruct((N, 1), jnp.float32)],
        compiler_params=_tc_params,
    )(state, W1, degr)

    aggp = _agg_kernel(y, src2, dst3)
    aggr = aggp.reshape(NC * NP, D)

    out = pl.pallas_call(
        _head_body,
        out_shape=jax.ShapeDtypeStruct((G, 1), jnp.float32),
        compiler_params=_tc_params,
    )(aggr, y, dinv, state, b1.reshape(1, D),
      Wl1, bl1.reshape(1, 64), Wl2, bl2.reshape(1, 64), Wl3, bl3.reshape(1, 1))
    return out[:, 0]


# speedup vs baseline: 1.3227x; 1.0786x over previous
import jax
import jax.numpy as jnp
from jax import lax
from jax.experimental import pallas as pl
from jax.experimental.pallas import tpu as pltpu
from jax.experimental.pallas import tpu_sc as plsc

N = 10000
E = 320000
D = 128
ACT = 16
G = N // ACT

NC = 2
NS = 16
NT = NC * NS
CHUNK = 128
CPT = 80
EPT = CPT * CHUNK
EP = NT * EPT
PAD_ROWS = 240
NP = N + PAD_ROWS
RPT = NP // NS
SUPERS = CPT // 2
WSUP = 4

_mesh = plsc.VectorSubcoreMesh(core_axis_name="c", subcore_axis_name="s")


def _deg_body(dst3, out, acc, dstb, ones_v, zbuf, dsem):
    c = lax.axis_index("c")
    s = lax.axis_index("s")
    w = s * NC + c
    o16 = jnp.ones((16,), jnp.float32)
    z16 = jnp.zeros((16,), jnp.float32)
    for jj in range(CHUNK // 16):
        ones_v[pl.ds(jj * 16, 16)] = o16
    for jj in range(RPT // 16):
        zbuf[pl.ds(jj * 16, 16)] = z16
    pltpu.sync_copy(zbuf, acc.at[pl.ds(s * RPT, RPT)])
    plsc.subcore_barrier()
    pltpu.sync_copy(dst3.at[w], dstb)

    def body(t, carry):
        for k in range(8):
            pltpu.async_copy(ones_v, acc.at[dstb.at[t * 8 + k]], dsem, add=True)
        for k in range(8):
            pltpu.make_async_copy(ones_v, acc.at[dstb.at[t * 8 + k]], dsem).wait()
        return carry

    lax.fori_loop(0, CPT // 8, body, 0)
    plsc.subcore_barrier()
    pltpu.sync_copy(acc.at[pl.ds(s * RPT, RPT)], out.at[c, pl.ds(s * RPT, RPT)])


_deg_kernel = pl.kernel(
    _deg_body,
    out_type=jax.ShapeDtypeStruct((NC, NP), jnp.float32),
    mesh=_mesh,
    scratch_types=[
        pltpu.VMEM_SHARED((NP,), jnp.float32),
        pltpu.VMEM((CPT, CHUNK), jnp.int32),
        pltpu.VMEM((CHUNK,), jnp.float32),
        pltpu.VMEM((RPT,), jnp.float32),
        pltpu.SemaphoreType.DMA,
    ],
)


def _agg_body(y_h, src2, dst3, out, acc, srcb, dstw, rows0, rows1,
              gsem0, gsem1, ssem0, ssem1, isem):
    c = lax.axis_index("c")
    s = lax.axis_index("s")
    w = s * NC + c
    z16 = jnp.zeros((16,), jnp.float32)

    def fire_g(it, buf, sem):
        pltpu.async_copy(y_h.at[srcb.at[pl.ds(it * CHUNK, CHUNK)]], buf, sem)

    def wait_g(it, buf, sem):
        pltpu.make_async_copy(y_h.at[srcb.at[pl.ds(it * CHUNK, CHUNK)]], buf,
                              sem).wait()

    def fire_s(it, buf, sem):
        pltpu.async_copy(buf, acc.at[dstw.at[it % 40]], sem, add=True)

    def wait_s(it, buf, sem):
        pltpu.make_async_copy(buf, acc.at[dstw.at[it % 40]], sem).wait()

    pltpu.sync_copy(src2.at[w], srcb)
    fire_g(0, rows1, gsem1)

    def zrow(i, carry):
        for jj in range(D // 16):
            rows0[i, pl.ds(jj * 16, 16)] = z16
        return carry

    lax.fori_loop(0, 16, zrow, 0)
    zsrc = rows0.at[pl.ds(0, 16)]
    for k in range(RPT // 16):
        pltpu.async_copy(zsrc, acc.at[pl.ds(s * RPT + k * 16, 16)], isem)
    for k in range(RPT // 16):
        pltpu.make_async_copy(zsrc, acc.at[pl.ds(s * RPT + k * 16, 16)],
                              isem).wait()
    pltpu.sync_copy(dst3.at[w, pl.ds(0, 40)], dstw)
    plsc.subcore_barrier()

    def body(t, carry):
        it0 = 2 * t
        it1 = it0 + 1

        @pl.when(t > 0)
        def _():
            wait_s(it1 - 2, rows0, ssem0)

        @pl.when(t == SUPERS // 2)
        def _():
            pltpu.sync_copy(dst3.at[w, pl.ds(40, 40)], dstw)

        fire_g(it1, rows0, gsem0)
        wait_g(it0, rows1, gsem1)
        fire_s(it0, rows1, ssem1)
        wait_g(it1, rows0, gsem0)
        wait_s(it0, rows1, ssem1)
        fire_s(it1, rows0, ssem0)

        @pl.when(t < SUPERS - 1)
        def _():
            fire_g(it0 + 2, rows1, gsem1)

        return carry

    lax.fori_loop(0, SUPERS, body, 0)
    wait_s(CPT - 1, rows0, ssem0)
    plsc.subcore_barrier()
    pltpu.sync_copy(acc.at[pl.ds(s * RPT, RPT)], out.at[c, pl.ds(s * RPT, RPT)])


_agg_kernel = pl.kernel(
    _agg_body,
    out_type=jax.ShapeDtypeStruct((NC, NP, D), jnp.float32),
    mesh=_mesh,
    scratch_types=[
        pltpu.VMEM_SHARED((NP, D), jnp.float32),
        pltpu.VMEM((EPT,), jnp.int32),
        pltpu.VMEM((40, CHUNK), jnp.int32),
        pltpu.VMEM((CHUNK, D), jnp.float32),
        pltpu.VMEM((CHUNK, D), jnp.float32),
        pltpu.SemaphoreType.DMA,
        pltpu.SemaphoreType.DMA,
        pltpu.SemaphoreType.DMA,
        pltpu.SemaphoreType.DMA,
        pltpu.SemaphoreType.DMA,
    ],
)


def _pre_body(state_ref, w1_ref, degr_ref, y_ref, dinv_ref):
    deg = degr_ref[0:N] + degr_ref[NP:NP + N] + 1.0
    dinv = lax.rsqrt(deg)
    xw = jnp.dot(state_ref[...], w1_ref[...], preferred_element_type=jnp.float32)
    y_ref[...] = xw * dinv
    dinv_ref[...] = dinv


def _head_body(aggr_ref, y_ref, dinv_ref, state_ref, b1_ref,
               wl1_ref, bl1_ref, wl2_ref, bl2_ref, wl3_ref, bl3_ref, out_ref):
    agg = aggr_ref[0:N] + aggr_ref[NP:NP + N] + y_ref[...]
    t = jnp.maximum(dinv_ref[...] * agg + b1_ref[...], 0.0) + state_ref[...]
    h = jnp.sum(t.reshape(G, ACT, D), axis=1)
    h = jnp.maximum(jnp.dot(h, wl1_ref[...], preferred_element_type=jnp.float32)
                    + bl1_ref[...], 0.0)
    h = jnp.maximum(jnp.dot(h, wl2_ref[...], preferred_element_type=jnp.float32)
                    + bl2_ref[...], 0.0)
    out_ref[...] = jnp.dot(h, wl3_ref[...], preferred_element_type=jnp.float32) \
        + bl3_ref[...]


_tc_params = pltpu.CompilerParams(vmem_limit_bytes=100 * 1024 * 1024)


@jax.jit
def kernel(state, edge_index, W1, b1, Wl1, bl1, Wl2, bl2, Wl3, bl3):
    pad = EP - E
    ar = jnp.arange(pad, dtype=jnp.int32)
    src_p = jnp.concatenate([edge_index[0], ar % 2048])
    dst_p = jnp.concatenate([edge_index[1], N + (ar % PAD_ROWS)])
    src2 = src_p.reshape(NT, EPT)
    dst3 = dst_p.reshape(NT, CPT, CHUNK)

    degp = _deg_kernel(dst3)
    degr = degp.reshape(NC * NP, 1)

    y, dinv = pl.pallas_call(
        _pre_body,
        out_shape=[jax.ShapeDtypeStruct((N, D), jnp.float32),
                   jax.ShapeDtypeStruct((N, 1), jnp.float32)],
        compiler_params=_tc_params,
    )(state, W1, degr)

    aggp = _agg_kernel(y, src2, dst3)
    aggr = aggp.reshape(NC * NP, D)

    out = pl.pallas_call(
        _head_body,
        out_shape=jax.ShapeDtypeStruct((G, 1), jnp.float32),
        compiler_params=_tc_params,
    )(aggr, y, dinv, state, b1.reshape(1, D),
      Wl1, bl1.reshape(1, 64), Wl2, bl2.reshape(1, 64), Wl3, bl3.reshape(1, 1))
    return out[:, 0]
